# Initial kernel scaffold; baseline (speedup 1.0000x reference)
import jax
import jax.numpy as jnp
from jax.experimental import pallas as pl


def kernel(table, rel_index):
    raise NotImplementedError("write your pallas kernel here")


# speedup vs baseline: 6.7333x; 6.7333x over previous
import functools

import jax
import jax.numpy as jnp
from jax import lax
from jax.experimental import pallas as pl
from jax.experimental.pallas import tpu as pltpu
from jax.experimental.pallas import tpu_sc as plsc

HEADS = 32
NUM_REL = 3969
NUM_REL_PAD = 3984
NC, NS, L = 2, 16, 16
NW = NC * NS


def _sc_gather(tableT, rel_flat, n_flat):
    per_w = n_flat // NW
    mesh = plsc.VectorSubcoreMesh(
        core_axis_name="c", subcore_axis_name="s", num_cores=NC, num_subcores=NS
    )

    @functools.partial(
        pl.kernel,
        out_type=jax.ShapeDtypeStruct((HEADS, n_flat), jnp.float32),
        mesh=mesh,
        compiler_params=pltpu.CompilerParams(needs_layout_passes=False),
        scratch_types=[
            pltpu.VMEM((per_w,), jnp.int32),
            pltpu.VMEM((NUM_REL_PAD,), jnp.float32),
            pltpu.VMEM((per_w,), jnp.float32),
            pltpu.SemaphoreType.DMA,
        ],
    )
    def k(tableT_hbm, rel_hbm, out_hbm, idx_v, row_v, out_v, sem):
        wid = lax.axis_index("s") * NC + lax.axis_index("c")
        base = wid * per_w
        pltpu.sync_copy(rel_hbm.at[pl.ds(base, per_w)], idx_v)

        def gather_head(h):
            pltpu.sync_copy(tableT_hbm.at[h], row_v)

            def body(i, _):
                iv = idx_v[pl.ds(i * L, L)]
                out_v[pl.ds(i * L, L)] = plsc.load_gather(row_v, [iv])
                return _

            lax.fori_loop(0, per_w // L, body, None)
            pltpu.async_copy(out_v, out_hbm.at[h, pl.ds(base, per_w)], sem).wait()

        for h in range(HEADS):
            gather_head(h)

    return k(tableT, rel_flat)


def kernel(table, rel_index):
    n = rel_index.shape[0]
    tableT = jnp.zeros((HEADS, NUM_REL_PAD), jnp.float32)
    tableT = tableT.at[:, :NUM_REL].set(table.T)
    rel_flat = rel_index.reshape(-1)
    out = _sc_gather(tableT, rel_flat, n * n)
    return out.reshape(HEADS, n, n)


# speedup vs baseline: 18.9458x; 2.8137x over previous
import functools

import jax
import jax.numpy as jnp
from jax import lax
from jax.experimental import pallas as pl
from jax.experimental.pallas import tpu as pltpu
from jax.experimental.pallas import tpu_sc as plsc

HEADS = 32
NUM_REL = 3969
NUM_REL_PAD = 4096
NC, NS, L = 2, 16, 16
NW = NC * NS


def _sc_gather(tableT, rel_flat, n_flat):
    per_w = n_flat // NW
    mesh = plsc.VectorSubcoreMesh(
        core_axis_name="c", subcore_axis_name="s", num_cores=NC, num_subcores=NS
    )

    @functools.partial(
        pl.kernel,
        out_type=jax.ShapeDtypeStruct((HEADS, n_flat), jnp.float32),
        mesh=mesh,
        compiler_params=pltpu.CompilerParams(needs_layout_passes=False),
        scratch_types=[
            pltpu.VMEM((per_w,), jnp.int32),
            pltpu.VMEM((2 * NUM_REL_PAD,), jnp.float32),
            pltpu.VMEM((2 * per_w,), jnp.float32),
            pltpu.SemaphoreType.DMA,
            pltpu.SemaphoreType.DMA,
            pltpu.SemaphoreType.DMA,
            pltpu.SemaphoreType.DMA,
        ],
    )
    def k(tableT_hbm, rel_hbm, out_hbm, idx_v, rows_v, out_v, rs0, rs1, os0, os1):
        wid = lax.axis_index("s") * NC + lax.axis_index("c")
        base = wid * per_w
        pltpu.sync_copy(rel_hbm.at[pl.ds(base, per_w)], idx_v)

        rsems, osems = (rs0, rs1), (os0, os1)
        row_desc = [None, None]
        out_desc = [None, None]
        row_desc[0] = pltpu.async_copy(
            tableT_hbm.at[0], rows_v.at[pl.ds(0, NUM_REL_PAD)], rsems[0]
        )
        for h in range(HEADS):
            b = h % 2
            row_desc[b].wait()
            if h + 1 < HEADS:
                nb = (h + 1) % 2
                row_desc[nb] = pltpu.async_copy(
                    tableT_hbm.at[h + 1],
                    rows_v.at[pl.ds(nb * NUM_REL_PAD, NUM_REL_PAD)],
                    rsems[nb],
                )
            if out_desc[b] is not None:
                out_desc[b].wait()
            row_off = b * NUM_REL_PAD
            out_off = b * per_w

            @plsc.parallel_loop(0, per_w // L, unroll=8)
            def gbody(i):
                iv = idx_v[pl.ds(i * L, L)] + row_off
                out_v[pl.ds(out_off + i * L, L)] = plsc.load_gather(rows_v, [iv])

            out_desc[b] = pltpu.async_copy(
                out_v.at[pl.ds(out_off, per_w)],
                out_hbm.at[h, pl.ds(base, per_w)],
                osems[b],
            )
        for d in out_desc:
            d.wait()

    return k(tableT, rel_flat)


def kernel(table, rel_index):
    n = rel_index.shape[0]
    tableT = jnp.zeros((HEADS, NUM_REL_PAD), jnp.float32)
    tableT = tableT.at[:, :NUM_REL].set(table.T)
    rel_flat = rel_index.reshape(-1)
    out = _sc_gather(tableT, rel_flat, n * n)
    return out.reshape(HEADS, n, n)


# speedup vs baseline: 34.3455x; 1.8128x over previous
import functools

import jax
import jax.numpy as jnp
from jax import lax
from jax.experimental import pallas as pl
from jax.experimental.pallas import tpu as pltpu
from jax.experimental.pallas import tpu_sc as plsc

HEADS = 32
NUM_REL = 3969
NUM_REL_PAD = 4096
NC, NS, L = 2, 16, 16
NW = NC * NS


def _sc_gather(tableT, rel_flat, n):
    n_flat = n * n
    per_w = n_flat // NW
    rows_w = per_w // n
    vpr = n // L
    mesh = plsc.VectorSubcoreMesh(
        core_axis_name="c", subcore_axis_name="s", num_cores=NC, num_subcores=NS
    )

    @functools.partial(
        pl.kernel,
        out_type=jax.ShapeDtypeStruct((HEADS, n, n), jnp.float32),
        mesh=mesh,
        compiler_params=pltpu.CompilerParams(needs_layout_passes=False),
        scratch_types=[
            pltpu.VMEM((per_w,), jnp.int32),
            pltpu.VMEM((2 * NUM_REL_PAD,), jnp.float32),
            pltpu.VMEM((2 * rows_w, n), jnp.float32),
            pltpu.SemaphoreType.DMA,
            pltpu.SemaphoreType.DMA,
            pltpu.SemaphoreType.DMA,
            pltpu.SemaphoreType.DMA,
        ],
    )
    def k(tableT_hbm, rel_hbm, out_hbm, idx_v, rows_v, out_v, rs0, rs1, os0, os1):
        wid = lax.axis_index("s") * NC + lax.axis_index("c")
        base = wid * per_w
        row_base = wid * rows_w
        pltpu.sync_copy(rel_hbm.at[pl.ds(base, per_w)], idx_v)

        rsems, osems = (rs0, rs1), (os0, os1)
        row_desc = [None, None]
        out_desc = [None, None]
        row_desc[0] = pltpu.async_copy(
            tableT_hbm.at[0], rows_v.at[pl.ds(0, NUM_REL_PAD)], rsems[0]
        )
        for h in range(HEADS):
            b = h % 2
            row_desc[b].wait()
            if h + 1 < HEADS:
                nb = (h + 1) % 2
                row_desc[nb] = pltpu.async_copy(
                    tableT_hbm.at[h + 1],
                    rows_v.at[pl.ds(nb * NUM_REL_PAD, NUM_REL_PAD)],
                    rsems[nb],
                )
            if out_desc[b] is not None:
                out_desc[b].wait()
            row_off = b * NUM_REL_PAD
            out_row = b * rows_w

            @plsc.parallel_loop(0, per_w // L, unroll=8)
            def gbody(i):
                r = i // vpr
                c = (i % vpr) * L
                iv = idx_v[pl.ds(i * L, L)] + row_off
                out_v[out_row + r, pl.ds(c, L)] = plsc.load_gather(rows_v, [iv])

            out_desc[b] = pltpu.async_copy(
                out_v.at[pl.ds(out_row, rows_w), :],
                out_hbm.at[h, pl.ds(row_base, rows_w), :],
                osems[b],
            )
        for d in out_desc:
            d.wait()

    return k(tableT, rel_flat)


def kernel(table, rel_index):
    n = rel_index.shape[0]
    tableT = jnp.zeros((HEADS, NUM_REL_PAD), jnp.float32)
    tableT = tableT.at[:, :NUM_REL].set(table.T)
    rel_flat = rel_index.reshape(-1)
    return _sc_gather(tableT, rel_flat, n)


# speedup vs baseline: 36.4891x; 1.0624x over previous
import functools

import jax
import jax.numpy as jnp
from jax import lax
from jax.experimental import pallas as pl
from jax.experimental.pallas import tpu as pltpu
from jax.experimental.pallas import tpu_sc as plsc

HEADS = 32
NUM_REL = 3969
NUM_REL_PAD = 4096
NC, NS, L = 2, 16, 16
HPC = HEADS // NC


def _sc_gather(tableT, rel_flat, n):
    n_flat = n * n
    per_s = n_flat // NS
    CH = n
    nch = per_s // CH
    assert nch % 2 == 0
    mesh = plsc.VectorSubcoreMesh(
        core_axis_name="c", subcore_axis_name="s", num_cores=NC, num_subcores=NS
    )

    @functools.partial(
        pl.kernel,
        out_type=jax.ShapeDtypeStruct((HEADS, n, n), jnp.float32),
        mesh=mesh,
        compiler_params=pltpu.CompilerParams(needs_layout_passes=False),
        scratch_types=[
            pltpu.VMEM((HPC * NUM_REL_PAD,), jnp.float32),
            pltpu.VMEM((2 * CH,), jnp.int32),
            pltpu.VMEM((2 * HPC, n), jnp.float32),
            pltpu.SemaphoreType.DMA,
            pltpu.SemaphoreType.DMA,
            pltpu.SemaphoreType.DMA,
            pltpu.SemaphoreType.DMA,
        ],
    )
    def k(tableT_hbm, rel_hbm, out_hbm, tab_v, idx_v, out_v, is0, is1, os0, os1):
        cid = lax.axis_index("c")
        sid = lax.axis_index("s")
        h0 = cid * HPC
        for r in range(HPC):
            pltpu.sync_copy(
                tableT_hbm.at[h0 + r], tab_v.at[pl.ds(r * NUM_REL_PAD, NUM_REL_PAD)]
            )
        base = sid * per_s
        row0 = sid * nch

        isems, osems = (is0, is1), (os0, os1)

        def idx_copy(a, b):
            return pltpu.make_async_copy(
                rel_hbm.at[pl.ds(base + a * CH, CH)],
                idx_v.at[pl.ds(b * CH, CH)],
                isems[b],
            )

        def out_drain(b):
            return pltpu.make_async_copy(
                out_hbm.at[0, pl.ds(0, HPC), :],
                out_v.at[pl.ds(b * HPC, HPC), :],
                osems[b],
            )

        idx_copy(0, 0).start()
        idx_copy(1, 1).start()

        def half(t2, a, b):
            idx_copy(a, b).wait()

            @pl.when(t2 > 0)
            def _():
                out_drain(b).wait()

            @plsc.parallel_loop(0, CH // L, unroll=2)
            def gbody(j):
                iv = idx_v[pl.ds(b * CH + j * L, L)]
                for h in range(HPC):
                    out_v[b * HPC + h, pl.ds(j * L, L)] = plsc.load_gather(
                        tab_v, [iv + h * NUM_REL_PAD]
                    )

            for h in range(HPC):
                pltpu.async_copy(
                    out_v.at[pl.ds(b * HPC + h, 1), :],
                    out_hbm.at[h0 + h, pl.ds(row0 + a, 1), :],
                    osems[b],
                )

            @pl.when(a + 2 < nch)
            def _():
                idx_copy(a + 2, b).start()

        def body(t2, _):
            half(t2, 2 * t2, 0)
            half(t2, 2 * t2 + 1, 1)
            return _

        lax.fori_loop(0, nch // 2, body, None)
        out_drain(0).wait()
        out_drain(1).wait()

    return k(tableT, rel_flat)


def kernel(table, rel_index):
    n = rel_index.shape[0]
    tableT = jnp.zeros((HEADS, NUM_REL_PAD), jnp.float32)
    tableT = tableT.at[:, :NUM_REL].set(table.T)
    rel_flat = rel_index.reshape(-1)
    return _sc_gather(tableT, rel_flat, n)


# speedup vs baseline: 42.6666x; 1.1693x over previous
import functools

import jax
import jax.numpy as jnp
from jax import lax
from jax.experimental import pallas as pl
from jax.experimental.pallas import tpu as pltpu
from jax.experimental.pallas import tpu_sc as plsc

HEADS = 32
NUM_REL = 3969
NUM_REL_PAD = 4096
NC, NS, L = 2, 16, 16
NW = NC * NS
HT = 4
NG = HEADS // HT
NQ = NW // NG
RR = 8


def _sc_gather(tableT, rel_flat, n):
    n_flat = n * n
    per_q = n_flat // NQ
    CH = RR * n
    nch = per_q // CH
    vpr = n // L
    assert nch % 2 == 0
    mesh = plsc.VectorSubcoreMesh(
        core_axis_name="c", subcore_axis_name="s", num_cores=NC, num_subcores=NS
    )

    @functools.partial(
        pl.kernel,
        out_type=jax.ShapeDtypeStruct((HEADS, n, n), jnp.float32),
        mesh=mesh,
        compiler_params=pltpu.CompilerParams(needs_layout_passes=False),
        scratch_types=[
            pltpu.VMEM((HT * NUM_REL_PAD,), jnp.float32),
            pltpu.VMEM((2 * CH,), jnp.int32),
            pltpu.VMEM((2 * HT * RR, n), jnp.float32),
            pltpu.SemaphoreType.DMA,
            pltpu.SemaphoreType.DMA,
            pltpu.SemaphoreType.DMA,
            pltpu.SemaphoreType.DMA,
        ],
    )
    def k(tableT_hbm, rel_hbm, out_hbm, tab_v, idx_v, out_v, is0, is1, os0, os1):
        wid = lax.axis_index("s") * NC + lax.axis_index("c")
        g = wid % NG
        q = wid // NG
        h0 = g * HT
        for r in range(HT):
            pltpu.sync_copy(
                tableT_hbm.at[h0 + r], tab_v.at[pl.ds(r * NUM_REL_PAD, NUM_REL_PAD)]
            )
        base = q * per_q
        row0 = q * (per_q // n)

        isems, osems = (is0, is1), (os0, os1)

        def idx_copy(a, b):
            return pltpu.make_async_copy(
                rel_hbm.at[pl.ds(base + a * CH, CH)],
                idx_v.at[pl.ds(b * CH, CH)],
                isems[b],
            )

        def out_drain(b):
            return pltpu.make_async_copy(
                out_hbm.at[0, pl.ds(0, HT * RR), :],
                out_v.at[pl.ds(b * HT * RR, HT * RR), :],
                osems[b],
            )

        idx_copy(0, 0).start()
        idx_copy(1, 1).start()

        def half(t2, a, b):
            idx_copy(a, b).wait()

            @pl.when(t2 > 0)
            def _():
                out_drain(b).wait()

            srow = b * HT * RR

            @plsc.parallel_loop(0, CH // L, unroll=4)
            def gbody(j):
                r = j // vpr
                c = (j % vpr) * L
                iv = idx_v[pl.ds(b * CH + j * L, L)]
                for h in range(HT):
                    out_v[srow + h * RR + r, pl.ds(c, L)] = plsc.load_gather(
                        tab_v, [iv + h * NUM_REL_PAD]
                    )

            for h in range(HT):
                pltpu.async_copy(
                    out_v.at[pl.ds(srow + h * RR, RR), :],
                    out_hbm.at[h0 + h, pl.ds(row0 + a * RR, RR), :],
                    osems[b],
                )

            @pl.when(a + 2 < nch)
            def _():
                idx_copy(a + 2, b).start()

        def body(t2, _):
            half(t2, 2 * t2, 0)
            half(t2, 2 * t2 + 1, 1)
            return _

        lax.fori_loop(0, nch // 2, body, None)
        out_drain(0).wait()
        out_drain(1).wait()

    return k(tableT, rel_flat)


def kernel(table, rel_index):
    n = rel_index.shape[0]
    tableT = jnp.zeros((HEADS, NUM_REL_PAD), jnp.float32)
    tableT = tableT.at[:, :NUM_REL].set(table.T)
    rel_flat = rel_index.reshape(-1)
    return _sc_gather(tableT, rel_flat, n)


# speedup vs baseline: 43.0753x; 1.0096x over previous
import functools

import jax
import jax.numpy as jnp
from jax import lax
from jax.experimental import pallas as pl
from jax.experimental.pallas import tpu as pltpu
from jax.experimental.pallas import tpu_sc as plsc

HEADS = 32
NUM_REL = 3969
NUM_REL_PAD = 4096
NC, NS, L = 2, 16, 16
NW = NC * NS
HT = 4
NG = HEADS // HT
NQ = NW // NG
RR = 8


def _sc_gather(tableT, rel_flat, n):
    n_flat = n * n
    per_q = n_flat // NQ
    CH = RR * n
    nch = per_q // CH
    vpr = n // L
    assert nch % 2 == 0
    mesh = plsc.VectorSubcoreMesh(
        core_axis_name="c", subcore_axis_name="s", num_cores=NC, num_subcores=NS
    )

    @functools.partial(
        pl.kernel,
        out_type=jax.ShapeDtypeStruct((HEADS, n, n), jnp.float32),
        mesh=mesh,
        compiler_params=pltpu.CompilerParams(needs_layout_passes=False),
        scratch_types=[
            pltpu.VMEM((HT * NUM_REL_PAD,), jnp.float32),
            pltpu.VMEM((2 * CH,), jnp.int32),
            pltpu.VMEM((2 * HT * RR, n), jnp.float32),
            pltpu.SemaphoreType.DMA,
            pltpu.SemaphoreType.DMA,
            pltpu.SemaphoreType.DMA,
            pltpu.SemaphoreType.DMA,
        ],
    )
    def k(tableT_hbm, rel_hbm, out_hbm, tab_v, idx_v, out_v, is0, is1, os0, os1):
        wid = lax.axis_index("s") * NC + lax.axis_index("c")
        g = wid % NG
        q = wid // NG
        h0 = g * HT
        base = q * per_q
        row0 = q * (per_q // n)

        isems, osems = (is0, is1), (os0, os1)

        def idx_copy(a, b):
            return pltpu.make_async_copy(
                rel_hbm.at[pl.ds(base + a * CH, CH)],
                idx_v.at[pl.ds(b * CH, CH)],
                isems[b],
            )

        def out_drain(b):
            return pltpu.make_async_copy(
                out_hbm.at[0, pl.ds(0, HT * RR), :],
                out_v.at[pl.ds(b * HT * RR, HT * RR), :],
                osems[b],
            )

        idx_copy(0, 0).start()
        idx_copy(1, 1).start()
        tab_descs = [
            pltpu.async_copy(
                tableT_hbm.at[h0 + r],
                tab_v.at[pl.ds(r * NUM_REL_PAD, NUM_REL_PAD)],
                os0,
            )
            for r in range(HT)
        ]
        for d in tab_descs:
            d.wait()

        def half(t2, a, b):
            idx_copy(a, b).wait()

            @pl.when(t2 > 0)
            def _():
                out_drain(b).wait()

            srow = b * HT * RR

            @plsc.parallel_loop(0, CH // L, unroll=4)
            def gbody(j):
                r = j // vpr
                c = (j % vpr) * L
                iv = idx_v[pl.ds(b * CH + j * L, L)]
                for h in range(HT):
                    out_v[srow + h * RR + r, pl.ds(c, L)] = plsc.load_gather(
                        tab_v, [iv + h * NUM_REL_PAD]
                    )

            for h in range(HT):
                pltpu.async_copy(
                    out_v.at[pl.ds(srow + h * RR, RR), :],
                    out_hbm.at[h0 + h, pl.ds(row0 + a * RR, RR), :],
                    osems[b],
                )

            @pl.when(a + 2 < nch)
            def _():
                idx_copy(a + 2, b).start()

        def body(t2, _):
            half(t2, 2 * t2, 0)
            half(t2, 2 * t2 + 1, 1)
            return _

        lax.fori_loop(0, nch // 2, body, None)
        out_drain(0).wait()
        out_drain(1).wait()

    return k(tableT, rel_flat)


def kernel(table, rel_index):
    n = rel_index.shape[0]
    tableT = jnp.zeros((HEADS, NUM_REL_PAD), jnp.float32)
    tableT = tableT.at[:, :NUM_REL].set(table.T)
    rel_flat = rel_index.reshape(-1)
    return _sc_gather(tableT, rel_flat, n)


# speedup vs baseline: 45.1227x; 1.0475x over previous
import functools

import jax
import jax.numpy as jnp
from jax import lax
from jax.experimental import pallas as pl
from jax.experimental.pallas import tpu as pltpu
from jax.experimental.pallas import tpu_sc as plsc

HEADS = 32
NUM_REL = 3969
NUM_REL_PAD = 4096
NC, NS, L = 2, 16, 16
NW = NC * NS
HT = 4
NG = HEADS // HT
NQ = NW // NG
RR = 8


def _sc_gather(tableT, rel_index, n):
    R_C0, R_C1 = 248, 264
    assert R_C0 % RR == 0 and R_C1 % RR == 0 and 2 * (R_C0 + R_C1) == n
    assert (R_C0 // RR) % 2 == 1 and (R_C1 // RR) % 2 == 1
    CH = RR * n
    vpr = n // L
    mesh = plsc.VectorSubcoreMesh(
        core_axis_name="c", subcore_axis_name="s", num_cores=NC, num_subcores=NS
    )

    @functools.partial(
        pl.kernel,
        out_type=jax.ShapeDtypeStruct((HEADS, n, n), jnp.float32),
        mesh=mesh,
        compiler_params=pltpu.CompilerParams(
            needs_layout_passes=False, use_tc_tiling_on_sc=True
        ),
        scratch_types=[
            pltpu.VMEM((HT * NUM_REL_PAD,), jnp.float32),
            pltpu.VMEM((2 * RR, n), jnp.int32),
            pltpu.VMEM((2 * HT * RR, n), jnp.float32),
            pltpu.SemaphoreType.DMA,
            pltpu.SemaphoreType.DMA,
            pltpu.SemaphoreType.DMA,
            pltpu.SemaphoreType.DMA,
        ],
    )
    def k(tableT_hbm, rel_hbm, out_hbm, tab_v, idx_v, out_v, is0, is1, os0, os1):
        cid = lax.axis_index("c")
        wid = lax.axis_index("s") * NC + cid
        g = wid // NQ
        sub = wid % NQ
        h0 = g * HT
        rows_t = R_C0 + cid * (R_C1 - R_C0)
        row0 = cid * 2 * R_C0 + (sub // 2) * rows_t
        nch = rows_t // RR

        isems, osems = (is0, is1), (os0, os1)

        def idx_copy(a, b):
            return pltpu.make_async_copy(
                rel_hbm.at[pl.ds(row0 + a * RR, RR), :],
                idx_v.at[pl.ds(b * RR, RR), :],
                isems[b],
            )

        def out_drain(b):
            return pltpu.make_async_copy(
                out_hbm.at[0, pl.ds(0, HT * RR), :],
                out_v.at[pl.ds(b * HT * RR, HT * RR), :],
                osems[b],
            )

        idx_copy(0, 0).start()
        idx_copy(1, 1).start()
        tab_descs = [
            pltpu.async_copy(
                tableT_hbm.at[h0 + r],
                tab_v.at[pl.ds(r * NUM_REL_PAD, NUM_REL_PAD)],
                os0,
            )
            for r in range(HT)
        ]
        for d in tab_descs:
            d.wait()

        def half(t2, a, b):
            idx_copy(a, b).wait()

            @pl.when(t2 > 0)
            def _():
                out_drain(b).wait()

            srow = b * HT * RR

            @plsc.parallel_loop(0, CH // L, unroll=4)
            def gbody(j):
                r = j // vpr
                c = (j % vpr) * L
                iv = idx_v[b * RR + r, pl.ds(c, L)]
                for h in range(HT):
                    out_v[srow + h * RR + r, pl.ds(c, L)] = plsc.load_gather(
                        tab_v, [iv + h * NUM_REL_PAD]
                    )

            for h in range(HT):
                pltpu.async_copy(
                    out_v.at[pl.ds(srow + h * RR, RR), :],
                    out_hbm.at[h0 + h, pl.ds(row0 + a * RR, RR), :],
                    osems[b],
                )

            @pl.when(a + 2 < nch)
            def _():
                idx_copy(a + 2, b).start()

        def body(t2, _):
            half(t2, 2 * t2, 0)
            half(t2, 2 * t2 + 1, 1)
            return _

        lax.fori_loop(0, nch // 2, body, None)
        half(nch // 2, nch - 1, 0)
        out_drain(0).wait()
        out_drain(1).wait()

    return k(tableT, rel_index)


def kernel(table, rel_index):
    n = rel_index.shape[0]
    tableT = jnp.pad(table.T, ((0, 0), (0, NUM_REL_PAD - NUM_REL)))
    return _sc_gather(tableT, rel_index, n)


# speedup vs baseline: 45.9963x; 1.0194x over previous
import functools

import jax
import jax.numpy as jnp
from jax import lax
from jax.experimental import pallas as pl
from jax.experimental.pallas import tpu as pltpu
from jax.experimental.pallas import tpu_sc as plsc

HEADS = 32
NUM_REL = 3969
NUM_REL_PAD = 4096
NC, NS, L = 2, 16, 16
NW = NC * NS
HT = 4
NG = HEADS // HT
NQ = NW // NG
RR = 8


def _sc_gather(tableT, rel_index, n):
    rows_q = n // NQ
    CH = RR * n
    nch = rows_q // RR
    vpr = n // L
    assert nch % 2 == 0
    mesh = plsc.VectorSubcoreMesh(
        core_axis_name="c", subcore_axis_name="s", num_cores=NC, num_subcores=NS
    )

    @functools.partial(
        pl.kernel,
        out_type=jax.ShapeDtypeStruct((HEADS, n, n), jnp.float32),
        mesh=mesh,
        compiler_params=pltpu.CompilerParams(
            needs_layout_passes=False, use_tc_tiling_on_sc=True
        ),
        scratch_types=[
            pltpu.VMEM((HT * NUM_REL_PAD,), jnp.float32),
            pltpu.VMEM((2 * RR, n), jnp.int32),
            pltpu.VMEM((2 * HT * RR, n), jnp.float32),
            pltpu.SemaphoreType.DMA,
            pltpu.SemaphoreType.DMA,
            pltpu.SemaphoreType.DMA,
            pltpu.SemaphoreType.DMA,
        ],
    )
    def k(tableT_hbm, rel_hbm, out_hbm, tab_v, idx_v, out_v, is0, is1, os0, os1):
        wid = lax.axis_index("s") * NC + lax.axis_index("c")
        g = wid % NG
        q = wid // NG
        h0 = g * HT
        row0 = q * rows_q

        isems, osems = (is0, is1), (os0, os1)

        def idx_copy(a, b):
            return pltpu.make_async_copy(
                rel_hbm.at[pl.ds(row0 + a * RR, RR), :],
                idx_v.at[pl.ds(b * RR, RR), :],
                isems[b],
            )

        def out_drain(b):
            return pltpu.make_async_copy(
                out_hbm.at[0, pl.ds(0, HT * RR), :],
                out_v.at[pl.ds(b * HT * RR, HT * RR), :],
                osems[b],
            )

        idx_copy(0, 0).start()
        idx_copy(1, 1).start()
        tab_descs = [
            pltpu.async_copy(
                tableT_hbm.at[h0 + r],
                tab_v.at[pl.ds(r * NUM_REL_PAD, NUM_REL_PAD)],
                os0,
            )
            for r in range(HT)
        ]
        for d in tab_descs:
            d.wait()

        def half(t2, a, b):
            idx_copy(a, b).wait()

            @pl.when(t2 > 0)
            def _():
                out_drain(b).wait()

            srow = b * HT * RR

            @plsc.parallel_loop(0, CH // L, unroll=4)
            def gbody(j):
                r = j // vpr
                c = (j % vpr) * L
                iv = idx_v[b * RR + r, pl.ds(c, L)]
                for h in range(HT):
                    out_v[srow + h * RR + r, pl.ds(c, L)] = plsc.load_gather(
                        tab_v, [iv + h * NUM_REL_PAD]
                    )

            for h in range(HT):
                pltpu.async_copy(
                    out_v.at[pl.ds(srow + h * RR, RR), :],
                    out_hbm.at[h0 + h, pl.ds(row0 + a * RR, RR), :],
                    osems[b],
                )

            @pl.when(a + 2 < nch)
            def _():
                idx_copy(a + 2, b).start()

        def body(t2, _):
            half(t2, 2 * t2, 0)
            half(t2, 2 * t2 + 1, 1)
            return _

        lax.fori_loop(0, nch // 2, body, None)
        out_drain(0).wait()
        out_drain(1).wait()

    return k(tableT, rel_index)


def kernel(table, rel_index):
    n = rel_index.shape[0]
    tableT = jnp.pad(table.T, ((0, 0), (0, NUM_REL_PAD - NUM_REL)))
    return _sc_gather(tableT, rel_index, n)
